# baseline (speedup 1.0000x reference)
import jax
import jax.numpy as jnp
from jax import lax
from jax.experimental import pallas as pl
from jax.experimental.pallas import tpu as pltpu
from jax.experimental.pallas import tpu_sc as plsc

N = 10000
E = 320000
D = 128
DH = D // 2
NT = 16
C = 128
NCHUNK = 162
EPT = NCHUNK * C
EP = NT * EPT
NP = 10240
RPT = NP // NT
NRF = N // NT
NBUF = 3
NIB = 6
AHEAD = 2
REC = 3 * C


def _sc_body(feat_h, pack_h, out, acc, feat_s,
             rows0, rows1, rows2, sem0, sem1, sem2, ssem0, ssem1, ssem2,
             ib0, ib1, ib2, ib3, ib4, ib5,
             isem0, isem1, isem2, isem3, isem4, isem5):
    c = lax.axis_index("c")
    s = lax.axis_index("s")
    bufs = (rows0, rows1, rows2)
    sems = (sem0, sem1, sem2)
    ssems = (ssem0, ssem1, ssem2)
    ibs = (ib0, ib1, ib2, ib3, ib4, ib5)
    isems = (isem0, isem1, isem2, isem3, isem4, isem5)

    pltpu.sync_copy(feat_h.at[c, pl.ds(s * NRF, NRF)],
                    feat_s.at[pl.ds(s * NRF, NRF)])

    zero16 = jnp.zeros((16,), jnp.float32)

    def _zrow(i, carry):
        for j in range(DH // 16):
            rows0[i, pl.ds(j * 16, 16)] = zero16
        return carry

    lax.fori_loop(0, C, _zrow, 0)
    for t in range(RPT // C):
        pltpu.sync_copy(rows0, acc.at[pl.ds(s * RPT + t * C, C)])
    plsc.subcore_barrier()

    def _gidx(m):
        return ibs[m].at[pl.ds(0, C)]

    def _sidx(m):
        return ibs[m].at[pl.ds(C, C)]

    def _scale(rows, ib):
        @plsc.parallel_loop(0, C, unroll=8,
                            carry=jnp.full((16,), 2 * C, jnp.int32))
        def _row(r, widx):
            w = plsc.bitcast(plsc.load_gather(ib, [widx]), jnp.float32)
            for q in range(DH // 16):
                rows[r, pl.ds(q * 16, 16)] = rows[r, pl.ds(q * 16, 16)] * w
            return widx + 1

    for j in range(NIB - 1):
        pltpu.async_copy(pack_h.at[s, j], ibs[j], isems[j])
    for k in range(AHEAD):
        pltpu.make_async_copy(pack_h.at[s, k], ibs[k], isems[k]).wait()
        pltpu.async_copy(feat_s.at[_gidx(k)], bufs[k], sems[k])

    def _hex(qi, carry):
        k0 = NIB * qi
        for u in range(NIB):
            k = k0 + u
            b = u % NBUF
            pb = (u - 1) % NBUF
            pm = (u - 1) % NIB
            nb = (u + AHEAD) % NBUF
            nm = (u + AHEAD) % NIB
            fm = (u + NIB - 1) % NIB

            @pl.when(k >= 1)
            def _drain_scatter():
                pltpu.make_async_copy(
                    bufs[pb], acc.at[_sidx(pm)], ssems[pb]).wait()

            @pl.when(k + NIB - 1 < NCHUNK)
            def _fetch_idx():
                pltpu.async_copy(
                    pack_h.at[s, k + NIB - 1], ibs[fm], isems[fm])

            @pl.when(k + AHEAD < NCHUNK)
            def _prefetch():
                pltpu.make_async_copy(
                    pack_h.at[s, k + AHEAD], ibs[nm], isems[nm]).wait()
                pltpu.async_copy(
                    feat_s.at[_gidx(nm)], bufs[nb], sems[nb])

            pltpu.make_async_copy(
                feat_s.at[_gidx(u)], bufs[b], sems[b]).wait()
            _scale(bufs[b], ibs[u])
            pltpu.async_copy(bufs[b], acc.at[_sidx(u)], ssems[b], add=True)
        return carry

    lax.fori_loop(0, NCHUNK // NIB, _hex, 0)
    pltpu.make_async_copy(
        bufs[(NCHUNK - 1) % NBUF], acc.at[_sidx((NCHUNK - 1) % NIB)],
        ssems[(NCHUNK - 1) % NBUF]).wait()
    plsc.subcore_barrier()

    for t in range(RPT // C):
        pltpu.sync_copy(acc.at[pl.ds(s * RPT + t * C, C)], rows0)
        pltpu.sync_copy(rows0, out.at[c, pl.ds(s * RPT + t * C, C)])


_sc_call = pl.kernel(
    _sc_body,
    out_type=jax.ShapeDtypeStruct((2, NP, DH), jnp.float32),
    mesh=plsc.VectorSubcoreMesh(core_axis_name="c", subcore_axis_name="s"),
    compiler_params=pltpu.CompilerParams(
        use_tc_tiling_on_sc=False, needs_layout_passes=False),
    scratch_types=(
        [pltpu.VMEM_SHARED((NP, DH), jnp.float32)]
        + [pltpu.VMEM_SHARED((N, DH), jnp.float32)]
        + [pltpu.VMEM((C, DH), jnp.float32)] * NBUF
        + [pltpu.SemaphoreType.DMA] * NBUF
        + [pltpu.SemaphoreType.DMA] * NBUF
        + [pltpu.VMEM((REC,), jnp.int32)] * NIB
        + [pltpu.SemaphoreType.DMA] * NIB
    ),
)


@jax.jit
def kernel(feat, edge_index, edge_weight):
    feat_h = jnp.stack([feat[:, :DH], feat[:, DH:]], axis=0)
    src = edge_index[0].astype(jnp.int32)
    dst = edge_index[1].astype(jnp.int32)
    pad = EP - E
    pad_idx = jnp.arange(pad, dtype=jnp.int32) % N
    src_p = jnp.concatenate([src, pad_idx])
    dst_p = jnp.concatenate([dst, pad_idx])
    w_bits = lax.bitcast_convert_type(
        jnp.concatenate([edge_weight, jnp.zeros((pad,), jnp.float32)]),
        jnp.int32)
    pack_h = jnp.stack(
        [dst_p.reshape(NT, NCHUNK, C), src_p.reshape(NT, NCHUNK, C),
         w_bits.reshape(NT, NCHUNK, C)], axis=2).reshape(NT, NCHUNK, REC)
    out2 = _sc_call(feat_h, pack_h)
    return jnp.concatenate([out2[0, :N], out2[1, :N]], axis=1)


# speedup vs baseline: 1.1693x; 1.1693x over previous
import jax
import jax.numpy as jnp
from jax import lax
from jax.experimental import pallas as pl
from jax.experimental.pallas import tpu as pltpu
from jax.experimental.pallas import tpu_sc as plsc

N = 10000
E = 320000
D = 128
NT = 16
C = 128
NCHUNK = 80
EPT = NCHUNK * C
EP = 2 * NT * EPT
NP = 10240
RPT = NP // NT
NBUF = 2
NIB = 4
REC = 3 * C


def _sc_body(feat_h, pack_h, out, acc,
             rows0, rows1, sem0, sem1, ssem0, ssem1,
             ib0, ib1, ib2, ib3, isem0, isem1, isem2, isem3):
    c = lax.axis_index("c")
    s = lax.axis_index("s")
    bufs = (rows0, rows1)
    sems = (sem0, sem1)
    ssems = (ssem0, ssem1)
    ibs = (ib0, ib1, ib2, ib3)
    isems = (isem0, isem1, isem2, isem3)

    zero16 = jnp.zeros((16,), jnp.float32)

    def _zrow(i, carry):
        for j in range(D // 16):
            rows0[i, pl.ds(j * 16, 16)] = zero16
        return carry

    lax.fori_loop(0, C, _zrow, 0)
    for t in range(RPT // C):
        pltpu.sync_copy(rows0, acc.at[pl.ds(s * RPT + t * C, C)])
    plsc.subcore_barrier()

    def _gidx(m):
        return ibs[m].at[pl.ds(0, C)]

    def _sidx(m):
        return ibs[m].at[pl.ds(C, C)]

    def _scale(rows, ib):
        @plsc.parallel_loop(0, C, unroll=8,
                            carry=jnp.full((16,), 2 * C, jnp.int32))
        def _row(r, widx):
            w = plsc.bitcast(plsc.load_gather(ib, [widx]), jnp.float32)
            for q in range(D // 16):
                rows[r, pl.ds(q * 16, 16)] = rows[r, pl.ds(q * 16, 16)] * w
            return widx + 1

    for j in range(NIB - 1):
        pltpu.async_copy(pack_h.at[c, s, j], ibs[j], isems[j])
    pltpu.make_async_copy(pack_h.at[c, s, 0], ibs[0], isems[0]).wait()
    pltpu.async_copy(feat_h.at[_gidx(0)], bufs[0], sems[0])

    def _quad(qi, carry):
        k0 = NIB * qi
        for u in range(NIB):
            k = k0 + u
            b = u % NBUF
            pb = (u - 1) % NBUF
            pm = (u - 1) % NIB
            nb = (u + 1) % NBUF
            nm = (u + 1) % NIB
            fm = (u + NIB - 1) % NIB

            @pl.when(k >= 1)
            def _drain_scatter():
                pltpu.make_async_copy(
                    bufs[pb], acc.at[_sidx(pm)], ssems[pb]).wait()

            @pl.when(k + NIB - 1 < NCHUNK)
            def _fetch_idx():
                pltpu.async_copy(
                    pack_h.at[c, s, k + NIB - 1], ibs[fm], isems[fm])

            @pl.when(k + 1 < NCHUNK)
            def _prefetch():
                pltpu.make_async_copy(
                    pack_h.at[c, s, k + 1], ibs[nm], isems[nm]).wait()
                pltpu.async_copy(
                    feat_h.at[_gidx(nm)], bufs[nb], sems[nb])

            pltpu.make_async_copy(
                feat_h.at[_gidx(u)], bufs[b], sems[b]).wait()
            _scale(bufs[b], ibs[u])
            pltpu.async_copy(bufs[b], acc.at[_sidx(u)], ssems[b], add=True)
        return carry

    lax.fori_loop(0, NCHUNK // NIB, _quad, 0)
    pltpu.make_async_copy(
        bufs[(NCHUNK - 1) % NBUF], acc.at[_sidx((NCHUNK - 1) % NIB)],
        ssems[(NCHUNK - 1) % NBUF]).wait()
    plsc.subcore_barrier()

    for t in range(RPT // C):
        pltpu.sync_copy(acc.at[pl.ds(s * RPT + t * C, C)], rows0)
        pltpu.sync_copy(rows0, out.at[c, pl.ds(s * RPT + t * C, C)])


_sc_call = pl.kernel(
    _sc_body,
    out_type=jax.ShapeDtypeStruct((2, NP, D), jnp.float32),
    mesh=plsc.VectorSubcoreMesh(core_axis_name="c", subcore_axis_name="s"),
    compiler_params=pltpu.CompilerParams(
        use_tc_tiling_on_sc=False, needs_layout_passes=False),
    scratch_types=(
        [pltpu.VMEM_SHARED((NP, D), jnp.float32)]
        + [pltpu.VMEM((C, D), jnp.float32)] * NBUF
        + [pltpu.SemaphoreType.DMA] * NBUF
        + [pltpu.SemaphoreType.DMA] * NBUF
        + [pltpu.VMEM((REC,), jnp.int32)] * NIB
        + [pltpu.SemaphoreType.DMA] * NIB
    ),
)


def _combine_body(p_ref, o_ref):
    o_ref[...] = p_ref[0] + p_ref[1]


_combine = pl.pallas_call(
    _combine_body,
    out_shape=jax.ShapeDtypeStruct((NP, D), jnp.float32),
)


@jax.jit
def kernel(feat, edge_index, edge_weight):
    src = edge_index[0].astype(jnp.int32)
    dst = edge_index[1].astype(jnp.int32)
    pad = EP - E
    pad_idx = jnp.arange(pad, dtype=jnp.int32) % N
    src_p = jnp.concatenate([src, pad_idx])
    dst_p = jnp.concatenate([dst, pad_idx])
    w_bits = lax.bitcast_convert_type(
        jnp.concatenate([edge_weight, jnp.zeros((pad,), jnp.float32)]),
        jnp.int32)
    pack_h = jnp.stack(
        [dst_p.reshape(2, NT, NCHUNK, C), src_p.reshape(2, NT, NCHUNK, C),
         w_bits.reshape(2, NT, NCHUNK, C)], axis=3).reshape(2, NT, NCHUNK, REC)
    partials = _sc_call(feat, pack_h)
    return _combine(partials)[:N]


# speedup vs baseline: 1.1780x; 1.0074x over previous
import jax
import jax.numpy as jnp
from jax import lax
from jax.experimental import pallas as pl
from jax.experimental.pallas import tpu as pltpu
from jax.experimental.pallas import tpu_sc as plsc

N = 10000
E = 320000
D = 128
NT = 16
C = 128
NCHUNK = 80
EPT = NCHUNK * C
EP = 2 * NT * EPT
NP = 10240
RPT = NP // NT
NBUF = 2
NIB = 4
REC = 3 * C


def _sc_body(feat_h, pack_h, out, acc,
             rows0, rows1, sem0, sem1, ssem0, ssem1,
             ib0, ib1, ib2, ib3, isem0, isem1, isem2, isem3):
    c = lax.axis_index("c")
    s = lax.axis_index("s")
    bufs = (rows0, rows1)
    sems = (sem0, sem1)
    ssems = (ssem0, ssem1)
    ibs = (ib0, ib1, ib2, ib3)
    isems = (isem0, isem1, isem2, isem3)

    zero16 = jnp.zeros((16,), jnp.float32)

    def _zrow(i, carry):
        for j in range(D // 16):
            rows0[i, pl.ds(j * 16, 16)] = zero16
        return carry

    lax.fori_loop(0, C, _zrow, 0)
    for t in range(RPT // C):
        pltpu.sync_copy(rows0, acc.at[pl.ds(s * RPT + t * C, C)])
    plsc.subcore_barrier()

    def _gidx(m):
        return ibs[m].at[pl.ds(0, C)]

    def _sidx(m):
        return ibs[m].at[pl.ds(C, C)]

    def _scale(rows, ib):
        @plsc.parallel_loop(0, C, unroll=8,
                            carry=jnp.full((16,), 2 * C, jnp.int32))
        def _row(r, widx):
            w = plsc.bitcast(plsc.load_gather(ib, [widx]), jnp.float32)
            for q in range(D // 16):
                rows[r, pl.ds(q * 16, 16)] = rows[r, pl.ds(q * 16, 16)] * w
            return widx + 1

    for j in range(NIB - 1):
        pltpu.async_copy(pack_h.at[c, s, j], ibs[j], isems[j])
    pltpu.make_async_copy(pack_h.at[c, s, 0], ibs[0], isems[0]).wait()
    pltpu.async_copy(feat_h.at[_gidx(0)], bufs[0], sems[0])

    def _quad(qi, carry):
        k0 = NIB * qi
        for u in range(NIB):
            k = k0 + u
            b = u % NBUF
            pb = (u - 1) % NBUF
            pm = (u - 1) % NIB
            nb = (u + 1) % NBUF
            nm = (u + 1) % NIB
            fm = (u + NIB - 1) % NIB

            @pl.when(k >= 1)
            def _drain_scatter():
                pltpu.make_async_copy(
                    bufs[pb], acc.at[_sidx(pm)], ssems[pb]).wait()

            @pl.when(k + NIB - 1 < NCHUNK)
            def _fetch_idx():
                pltpu.async_copy(
                    pack_h.at[c, s, k + NIB - 1], ibs[fm], isems[fm])

            @pl.when(k + 1 < NCHUNK)
            def _prefetch():
                pltpu.make_async_copy(
                    pack_h.at[c, s, k + 1], ibs[nm], isems[nm]).wait()
                pltpu.async_copy(
                    feat_h.at[_gidx(nm)], bufs[nb], sems[nb])

            pltpu.make_async_copy(
                feat_h.at[_gidx(u)], bufs[b], sems[b]).wait()
            _scale(bufs[b], ibs[u])
            pltpu.async_copy(bufs[b], acc.at[_sidx(u)], ssems[b], add=True)
        return carry

    lax.fori_loop(0, NCHUNK // NIB, _quad, 0)
    pltpu.make_async_copy(
        bufs[(NCHUNK - 1) % NBUF], acc.at[_sidx((NCHUNK - 1) % NIB)],
        ssems[(NCHUNK - 1) % NBUF]).wait()
    plsc.subcore_barrier()

    for t in range(RPT // C):
        pltpu.sync_copy(acc.at[pl.ds(s * RPT + t * C, C)], rows0)
        pltpu.sync_copy(rows0, out.at[c, pl.ds(s * RPT + t * C, C)])


_sc_call = pl.kernel(
    _sc_body,
    out_type=jax.ShapeDtypeStruct((2, NP, D), jnp.float32),
    mesh=plsc.VectorSubcoreMesh(core_axis_name="c", subcore_axis_name="s"),
    compiler_params=pltpu.CompilerParams(
        use_tc_tiling_on_sc=False, needs_layout_passes=False),
    scratch_types=(
        [pltpu.VMEM_SHARED((NP, D), jnp.float32)]
        + [pltpu.VMEM((C, D), jnp.float32)] * NBUF
        + [pltpu.SemaphoreType.DMA] * NBUF
        + [pltpu.SemaphoreType.DMA] * NBUF
        + [pltpu.VMEM((REC,), jnp.int32)] * NIB
        + [pltpu.SemaphoreType.DMA] * NIB
    ),
)


def _combine_body(p_ref, o_ref):
    o_ref[...] = p_ref[0] + p_ref[1]


_CB = 1000
_combine = pl.pallas_call(
    _combine_body,
    grid=(N // _CB,),
    in_specs=[pl.BlockSpec((2, _CB, D), lambda i: (0, i, 0))],
    out_specs=pl.BlockSpec((_CB, D), lambda i: (i, 0)),
    out_shape=jax.ShapeDtypeStruct((N, D), jnp.float32),
)


@jax.jit
def kernel(feat, edge_index, edge_weight):
    src = edge_index[0].astype(jnp.int32)
    dst = edge_index[1].astype(jnp.int32)
    pad = EP - E
    pad_idx = jnp.arange(pad, dtype=jnp.int32) % N
    src_p = jnp.concatenate([src, pad_idx])
    dst_p = jnp.concatenate([dst, pad_idx])
    w_bits = lax.bitcast_convert_type(
        jnp.concatenate([edge_weight, jnp.zeros((pad,), jnp.float32)]),
        jnp.int32)
    pack_h = jnp.stack(
        [dst_p.reshape(2, NT, NCHUNK, C), src_p.reshape(2, NT, NCHUNK, C),
         w_bits.reshape(2, NT, NCHUNK, C)], axis=3).reshape(2, NT, NCHUNK, REC)
    partials = _sc_call(feat, pack_h)
    return _combine(partials)


# speedup vs baseline: 1.1856x; 1.0065x over previous
import jax
import jax.numpy as jnp
from jax import lax
from jax.experimental import pallas as pl
from jax.experimental.pallas import tpu as pltpu
from jax.experimental.pallas import tpu_sc as plsc

N = 10000
E = 320000
D = 128
NT = 16
C = 128
NCHUNK = 80
EPT = NCHUNK * C
EP = 2 * NT * EPT
NP = 10240
RPT = NP // NT
NBUF = 2
NIB = 4
REC = 3 * C


def _sc_body(feat_h, pack_h, out, acc,
             rows0, rows1, sem0, sem1, ssem0, ssem1,
             ib0, ib1, ib2, ib3, isem0, isem1, isem2, isem3):
    c = lax.axis_index("c")
    s = lax.axis_index("s")
    bufs = (rows0, rows1)
    sems = (sem0, sem1)
    ssems = (ssem0, ssem1)
    ibs = (ib0, ib1, ib2, ib3)
    isems = (isem0, isem1, isem2, isem3)

    for j in range(NIB - 1):
        pltpu.async_copy(pack_h.at[c, s, j], ibs[j], isems[j])

    zero16 = jnp.zeros((16,), jnp.float32)

    def _zrow(i, carry):
        for j in range(D // 16):
            rows0[i, pl.ds(j * 16, 16)] = zero16
        return carry

    lax.fori_loop(0, C, _zrow, 0)
    for t in range(RPT // C):
        pltpu.async_copy(rows0, acc.at[pl.ds(s * RPT + t * C, C)], ssem0)
    for t in range(RPT // C):
        pltpu.make_async_copy(
            rows0, acc.at[pl.ds(s * RPT + t * C, C)], ssem0).wait()
    plsc.subcore_barrier()

    def _gidx(m):
        return ibs[m].at[pl.ds(0, C)]

    def _sidx(m):
        return ibs[m].at[pl.ds(C, C)]

    def _scale(rows, ib):
        @plsc.parallel_loop(0, C, unroll=8,
                            carry=jnp.full((16,), 2 * C, jnp.int32))
        def _row(r, widx):
            w = plsc.bitcast(plsc.load_gather(ib, [widx]), jnp.float32)
            for q in range(D // 16):
                rows[r, pl.ds(q * 16, 16)] = rows[r, pl.ds(q * 16, 16)] * w
            return widx + 1

    pltpu.make_async_copy(pack_h.at[c, s, 0], ibs[0], isems[0]).wait()
    pltpu.async_copy(feat_h.at[_gidx(0)], bufs[0], sems[0])

    def _quad(qi, carry):
        k0 = NIB * qi
        for u in range(NIB):
            k = k0 + u
            b = u % NBUF
            pb = (u - 1) % NBUF
            pm = (u - 1) % NIB
            nb = (u + 1) % NBUF
            nm = (u + 1) % NIB
            fm = (u + NIB - 1) % NIB

            @pl.when(k >= 1)
            def _drain_scatter():
                pltpu.make_async_copy(
                    bufs[pb], acc.at[_sidx(pm)], ssems[pb]).wait()

            @pl.when(k + NIB - 1 < NCHUNK)
            def _fetch_idx():
                pltpu.async_copy(
                    pack_h.at[c, s, k + NIB - 1], ibs[fm], isems[fm])

            @pl.when(k + 1 < NCHUNK)
            def _prefetch():
                pltpu.make_async_copy(
                    pack_h.at[c, s, k + 1], ibs[nm], isems[nm]).wait()
                pltpu.async_copy(
                    feat_h.at[_gidx(nm)], bufs[nb], sems[nb])

            pltpu.make_async_copy(
                feat_h.at[_gidx(u)], bufs[b], sems[b]).wait()
            _scale(bufs[b], ibs[u])
            pltpu.async_copy(bufs[b], acc.at[_sidx(u)], ssems[b], add=True)
        return carry

    lax.fori_loop(0, NCHUNK // NIB, _quad, 0)
    pltpu.make_async_copy(
        bufs[(NCHUNK - 1) % NBUF], acc.at[_sidx((NCHUNK - 1) % NIB)],
        ssems[(NCHUNK - 1) % NBUF]).wait()
    plsc.subcore_barrier()

    pltpu.sync_copy(acc.at[pl.ds(s * RPT, RPT)],
                    out.at[c, pl.ds(s * RPT, RPT)])


_sc_call = pl.kernel(
    _sc_body,
    out_type=jax.ShapeDtypeStruct((2, NP, D), jnp.float32),
    mesh=plsc.VectorSubcoreMesh(core_axis_name="c", subcore_axis_name="s"),
    compiler_params=pltpu.CompilerParams(
        use_tc_tiling_on_sc=False, needs_layout_passes=False),
    scratch_types=(
        [pltpu.VMEM_SHARED((NP, D), jnp.float32)]
        + [pltpu.VMEM((C, D), jnp.float32)] * NBUF
        + [pltpu.SemaphoreType.DMA] * NBUF
        + [pltpu.SemaphoreType.DMA] * NBUF
        + [pltpu.VMEM((REC,), jnp.int32)] * NIB
        + [pltpu.SemaphoreType.DMA] * NIB
    ),
)


def _combine_body(p_ref, o_ref):
    o_ref[...] = p_ref[0] + p_ref[1]


_CB = 1000
_combine = pl.pallas_call(
    _combine_body,
    grid=(N // _CB,),
    in_specs=[pl.BlockSpec((2, _CB, D), lambda i: (0, i, 0))],
    out_specs=pl.BlockSpec((_CB, D), lambda i: (i, 0)),
    out_shape=jax.ShapeDtypeStruct((N, D), jnp.float32),
)


@jax.jit
def kernel(feat, edge_index, edge_weight):
    src = edge_index[0].astype(jnp.int32)
    dst = edge_index[1].astype(jnp.int32)
    pad = EP - E
    pad_idx = jnp.arange(pad, dtype=jnp.int32) % N
    src_p = jnp.concatenate([src, pad_idx])
    dst_p = jnp.concatenate([dst, pad_idx])
    w_bits = lax.bitcast_convert_type(
        jnp.concatenate([edge_weight, jnp.zeros((pad,), jnp.float32)]),
        jnp.int32)
    pack_h = jnp.stack(
        [dst_p.reshape(2, NT, NCHUNK, C), src_p.reshape(2, NT, NCHUNK, C),
         w_bits.reshape(2, NT, NCHUNK, C)], axis=3).reshape(2, NT, NCHUNK, REC)
    partials = _sc_call(feat, pack_h)
    return _combine(partials)


# speedup vs baseline: 1.2189x; 1.0280x over previous
import jax
import jax.numpy as jnp
from jax import lax
from jax.experimental import pallas as pl
from jax.experimental.pallas import tpu as pltpu
from jax.experimental.pallas import tpu_sc as plsc

N = 10000
E = 320000
D = 128
NT = 16
C = 128
NCHUNK = 80
EPT = NCHUNK * C
EP = 2 * NT * EPT
NP = 10240
RPT = NP // NT
NBUF = 2
NIB = 4
REC = 3 * C


def _sc_body(feat_h, gi_h, si_h, wb_h, out, acc,
             rows0, rows1, sem0, sem1, ssem0, ssem1,
             gb0, gb1, gb2, gb3, sb0, sb1, sb2, sb3,
             wv0, wv1, wv2, wv3, isem0, isem1, isem2, isem3):
    c = lax.axis_index("c")
    s = lax.axis_index("s")
    bufs = (rows0, rows1)
    sems = (sem0, sem1)
    ssems = (ssem0, ssem1)
    gbs = (gb0, gb1, gb2, gb3)
    sbs = (sb0, sb1, sb2, sb3)
    wvs = (wv0, wv1, wv2, wv3)
    isems = (isem0, isem1, isem2, isem3)

    def _fetch_idx_rec(m, k):
        pltpu.async_copy(gi_h.at[c, s, k], gbs[m], isems[m])
        pltpu.async_copy(si_h.at[c, s, k], sbs[m], isems[m])
        pltpu.async_copy(wb_h.at[c, s, k], wvs[m], isems[m])

    def _wait_idx_rec(m, k):
        pltpu.make_async_copy(gi_h.at[c, s, k], gbs[m], isems[m]).wait()
        pltpu.make_async_copy(si_h.at[c, s, k], sbs[m], isems[m]).wait()
        pltpu.make_async_copy(wb_h.at[c, s, k], wvs[m], isems[m]).wait()

    for j in range(NIB - 1):
        _fetch_idx_rec(j, j)

    zero16 = jnp.zeros((16,), jnp.float32)

    def _zrow(i, carry):
        for j in range(D // 16):
            rows0[i, pl.ds(j * 16, 16)] = zero16
        return carry

    lax.fori_loop(0, C, _zrow, 0)
    for t in range(RPT // C):
        pltpu.async_copy(rows0, acc.at[pl.ds(s * RPT + t * C, C)], ssem0)
    for t in range(RPT // C):
        pltpu.make_async_copy(
            rows0, acc.at[pl.ds(s * RPT + t * C, C)], ssem0).wait()
    plsc.subcore_barrier()

    def _gidx(m):
        return gbs[m].at[pl.ds(0, C)]

    def _sidx(m):
        return sbs[m].at[pl.ds(0, C)]

    def _scale(rows, wv):
        @plsc.parallel_loop(0, C, unroll=8,
                            carry=jnp.full((16,), 0, jnp.int32))
        def _row(r, widx):
            w = plsc.bitcast(plsc.load_gather(wv, [widx]), jnp.float32)
            for q in range(D // 16):
                rows[r, pl.ds(q * 16, 16)] = rows[r, pl.ds(q * 16, 16)] * w
            return widx + 1

    _wait_idx_rec(0, 0)
    pltpu.async_copy(feat_h.at[_gidx(0)], bufs[0], sems[0])

    def _quad(qi, carry):
        k0 = NIB * qi
        for u in range(NIB):
            k = k0 + u
            b = u % NBUF
            pb = (u - 1) % NBUF
            pm = (u - 1) % NIB
            nb = (u + 1) % NBUF
            nm = (u + 1) % NIB
            fm = (u + NIB - 1) % NIB

            @pl.when(k >= 1)
            def _drain_scatter():
                pltpu.make_async_copy(
                    bufs[pb], acc.at[_sidx(pm)], ssems[pb]).wait()

            @pl.when(k + NIB - 1 < NCHUNK)
            def _fetch_idx():
                _fetch_idx_rec(fm, k + NIB - 1)

            @pl.when(k + 1 < NCHUNK)
            def _prefetch():
                _wait_idx_rec(nm, k + 1)
                pltpu.async_copy(
                    feat_h.at[_gidx(nm)], bufs[nb], sems[nb])

            pltpu.make_async_copy(
                feat_h.at[_gidx(u)], bufs[b], sems[b]).wait()
            _scale(bufs[b], wvs[u])
            pltpu.async_copy(bufs[b], acc.at[_sidx(u)], ssems[b], add=True)
        return carry

    lax.fori_loop(0, NCHUNK // NIB, _quad, 0)
    pltpu.make_async_copy(
        bufs[(NCHUNK - 1) % NBUF], acc.at[_sidx((NCHUNK - 1) % NIB)],
        ssems[(NCHUNK - 1) % NBUF]).wait()
    plsc.subcore_barrier()

    pltpu.sync_copy(acc.at[pl.ds(s * RPT, RPT)],
                    out.at[c, pl.ds(s * RPT, RPT)])


_sc_call = pl.kernel(
    _sc_body,
    out_type=jax.ShapeDtypeStruct((2, NP, D), jnp.float32),
    mesh=plsc.VectorSubcoreMesh(core_axis_name="c", subcore_axis_name="s"),
    compiler_params=pltpu.CompilerParams(
        use_tc_tiling_on_sc=False, needs_layout_passes=False),
    scratch_types=(
        [pltpu.VMEM_SHARED((NP, D), jnp.float32)]
        + [pltpu.VMEM((C, D), jnp.float32)] * NBUF
        + [pltpu.SemaphoreType.DMA] * NBUF
        + [pltpu.SemaphoreType.DMA] * NBUF
        + [pltpu.VMEM((C,), jnp.int32)] * (3 * NIB)
        + [pltpu.SemaphoreType.DMA] * NIB
    ),
)


def _combine_body(p_ref, o_ref):
    o_ref[...] = p_ref[0] + p_ref[1]


_CB = 1000
_combine = pl.pallas_call(
    _combine_body,
    grid=(N // _CB,),
    in_specs=[pl.BlockSpec((2, _CB, D), lambda i: (0, i, 0))],
    out_specs=pl.BlockSpec((_CB, D), lambda i: (i, 0)),
    out_shape=jax.ShapeDtypeStruct((N, D), jnp.float32),
)


@jax.jit
def kernel(feat, edge_index, edge_weight):
    src = edge_index[0].astype(jnp.int32)
    dst = edge_index[1].astype(jnp.int32)
    pad = EP - E
    pad_idx = jnp.arange(pad, dtype=jnp.int32) % N
    src_p = jnp.concatenate([src, pad_idx])
    dst_p = jnp.concatenate([dst, pad_idx])
    w_bits = lax.bitcast_convert_type(
        jnp.concatenate([edge_weight, jnp.zeros((pad,), jnp.float32)]),
        jnp.int32)
    gi_h = dst_p.reshape(2, NT, NCHUNK, C)
    si_h = src_p.reshape(2, NT, NCHUNK, C)
    wb_h = w_bits.reshape(2, NT, NCHUNK, C)
    partials = _sc_call(feat, gi_h, si_h, wb_h)
    return _combine(partials)
